# Initial kernel scaffold; baseline (speedup 1.0000x reference)
import jax
import jax.numpy as jnp
from jax.experimental import pallas as pl


def kernel(edge_index_g2, edge_type_g2, edge_index_g1, all_node_embedding, basis1, comp1, root1, bias1, basis2, comp2, root2, bias2):
    raise NotImplementedError("write your pallas kernel here")


# speedup vs baseline: 1.2868x; 1.2868x over previous
import functools

import jax
import jax.numpy as jnp
from jax import lax
from jax.experimental import pallas as pl
from jax.experimental.pallas import tpu as pltpu

_N1 = 110000
_N2 = 100000
_D = 128
_H = 64
_C = 8
_R = 8
_B = 4
_E = 320000

_NBLK = 800
_EBLK = 2000


def _relu_mm_body(x_ref, w_ref, o_ref):
    x = jnp.maximum(x_ref[...], 0.0)
    o_ref[...] = jnp.dot(x, w_ref[...], preferred_element_type=jnp.float32)


def _relu_mm(x, w, nblk):
    n, d = x.shape
    k = w.shape[1]
    return pl.pallas_call(
        _relu_mm_body,
        grid=(n // nblk,),
        in_specs=[
            pl.BlockSpec((nblk, d), lambda i: (i, 0)),
            pl.BlockSpec((d, k), lambda i: (0, 0)),
        ],
        out_specs=pl.BlockSpec((nblk, k), lambda i: (i, 0)),
        out_shape=jax.ShapeDtypeStruct((n, k), jnp.float32),
    )(x, w)


def _edge_combine_body(nb, hw, hb_ref, w_ref, o_ref):
    w = w_ref[...]
    acc = w[:, 0:1] * hb_ref[:, 0:hw]
    for b in range(1, nb):
        acc = acc + w[:, b:b + 1] * hb_ref[:, b * hw:(b + 1) * hw]
    o_ref[...] = acc


def _edge_combine(hb_src, w, hw):
    e = hb_src.shape[0]
    nb = w.shape[1]
    return pl.pallas_call(
        functools.partial(_edge_combine_body, nb, hw),
        grid=(e // _EBLK,),
        in_specs=[
            pl.BlockSpec((_EBLK, nb * hw), lambda i: (i, 0)),
            pl.BlockSpec((_EBLK, nb), lambda i: (i, 0)),
        ],
        out_specs=pl.BlockSpec((_EBLK, hw), lambda i: (i, 0)),
        out_shape=jax.ShapeDtypeStruct((e, hw), jnp.float32),
    )(hb_src, w)


def _softmax_body(z_ref, o_ref):
    z = z_ref[...]
    z = z - jnp.max(z, axis=1, keepdims=True)
    ez = jnp.exp(z)
    o_ref[...] = ez / jnp.sum(ez, axis=1, keepdims=True)


def _softmax(z):
    n, c = z.shape
    return pl.pallas_call(
        _softmax_body,
        grid=(n // _NBLK,),
        in_specs=[pl.BlockSpec((_NBLK, c), lambda i: (i, 0))],
        out_specs=pl.BlockSpec((_NBLK, c), lambda i: (i, 0)),
        out_shape=jax.ShapeDtypeStruct((n, c), jnp.float32),
    )(z)


def kernel(edge_index_g2, edge_type_g2, edge_index_g1, all_node_embedding,
           basis1, comp1, root1, bias1, basis2, comp2, root2, bias2):
    src1 = edge_index_g1[0]
    dst1 = edge_index_g1[1]
    src2 = edge_index_g2[0]
    dst2 = edge_index_g2[1]
    et = edge_type_g2

    ones = jnp.ones((_E,), dtype=jnp.float32)

    gathered = jnp.take(all_node_embedding, src1, axis=0)
    agg0 = jax.ops.segment_sum(gathered, dst1, num_segments=_N1)
    deg = jax.ops.segment_sum(ones, dst1, num_segments=_N1)
    x_pre = agg0[:_N2] / jnp.maximum(deg[:_N2], 1.0)[:, None]

    keyid = dst2 * _R + et
    cnt = jax.ops.segment_sum(ones, keyid, num_segments=_N2 * _R)
    norm = 1.0 / jnp.maximum(jnp.take(cnt, keyid), 1.0)
    w1 = jnp.take(comp1, et, axis=0) * norm[:, None]
    w2 = jnp.take(comp2, et, axis=0) * norm[:, None]

    wmat1 = jnp.concatenate(
        [jnp.transpose(basis1, (1, 0, 2)).reshape(_D, _B * _H), root1], axis=1)
    wmat1 = jnp.pad(wmat1, ((0, 0), (0, 384 - _B * _H - _H)))
    hb1 = _relu_mm(x_pre, wmat1, _NBLK)
    rt1 = hb1[:, _B * _H:_B * _H + _H]
    hb1_src = jnp.take(hb1[:, :_B * _H], src2, axis=0)
    msg1 = _edge_combine(hb1_src, w1, _H)
    agg1 = jax.ops.segment_sum(msg1, dst2, num_segments=_N2)
    pre2 = agg1 + rt1 + bias1[None, :]

    wmat2 = jnp.concatenate(
        [jnp.transpose(basis2, (1, 0, 2)).reshape(_H, _B * _C), root2], axis=1)
    wmat2 = jnp.pad(wmat2, ((0, 0), (0, 128 - _B * _C - _C)))
    hb2 = _relu_mm(pre2, wmat2, _NBLK)
    rt2 = hb2[:, _B * _C:_B * _C + _C]
    hb2_src = jnp.take(hb2[:, :_B * _C], src2, axis=0)
    msg2 = _edge_combine(hb2_src, w2, _C)
    agg2 = jax.ops.segment_sum(msg2, dst2, num_segments=_N2)
    z = agg2 + rt2 + bias2[None, :]

    return _softmax(z)
